# baseline (speedup 1.0000x reference)
import functools

import jax
import jax.numpy as jnp
from jax import lax
from jax.experimental import pallas as pl
from jax.experimental.pallas import tpu as pltpu
from jax.experimental.pallas import tpu_sc as plsc

VOCAB_ROWS = 1_000_000
EMB_DIM = 64
SEQ = 50
BATCH = 16384
NUM_CORES = 2
NUM_SUBCORES = 16
NUM_WORKERS = NUM_CORES * NUM_SUBCORES
B_PER_W = BATCH // NUM_WORKERS
CHUNK = 128
NBLK = B_PER_W // CHUNK


@jax.jit
def _sc_gather_t(table, idx_flat):
  mesh = plsc.VectorSubcoreMesh(core_axis_name="c", subcore_axis_name="s")

  @functools.partial(
      pl.kernel,
      out_type=jax.ShapeDtypeStruct((SEQ, EMB_DIM, BATCH), jnp.float32),
      mesh=mesh,
      scratch_types=[
          pltpu.VMEM((B_PER_W * SEQ,), jnp.int32),
          pltpu.VMEM((2, B_PER_W), jnp.int32),
          pltpu.VMEM((NBLK, CHUNK, EMB_DIM), jnp.float32),
          pltpu.VMEM((2, EMB_DIM, B_PER_W), jnp.float32),
          [pltpu.SemaphoreType.DMA] * NBLK,
          [pltpu.SemaphoreType.DMA] * 2,
      ],
      compiler_params=pltpu.CompilerParams(
          use_tc_tiling_on_sc=False, needs_layout_passes=False
      ),
  )
  def body(table_hbm, idx_hbm, out_hbm, x_v, idx_v, bufa, bufb, gsems, wsems):
    wid = lax.axis_index("s") * NUM_CORES + lax.axis_index("c")
    b0 = wid * B_PER_W
    pltpu.sync_copy(idx_hbm.at[pl.ds(b0 * SEQ, B_PER_W * SEQ)], x_v)

    iota = lax.iota(jnp.int32, 16)
    iota_s = iota * SEQ
    rows_g = [iota + 16 * g for g in range(8)]

    def build_idx(s, slot):
      def step(g, carry):
        v = plsc.load_gather(x_v, [iota_s + (g * 16 * SEQ + s)])
        idx_v[slot, pl.ds(g * 16, 16)] = v
        return carry
      lax.fori_loop(0, B_PER_W // 16, step, 0)

    def gather(s, slot, blk):
      pltpu.async_copy(
          table_hbm.at[idx_v.at[slot].at[pl.ds(blk * CHUNK, CHUNK)]],
          bufa.at[blk],
          gsems[blk],
      )

    def wait_gather(blk):
      pltpu.make_async_copy(
          table_hbm.at[idx_v.at[0].at[pl.ds(0, CHUNK)]],
          bufa.at[blk],
          gsems[blk],
      ).wait()

    def transpose(blk, sb):
      src = bufa.at[blk]
      def step(d, carry):
        col = jnp.full((16,), d, jnp.int32)
        for g in range(8):
          v = plsc.load_gather(src, [rows_g[g], col])
          bufb[sb, d, pl.ds(blk * CHUNK + g * 16, 16)] = v
        return carry
      lax.fori_loop(0, EMB_DIM, step, 0)

    def write(s, sb):
      pltpu.async_copy(
          bufb.at[sb], out_hbm.at[s, :, pl.ds(b0, B_PER_W)], wsems[sb]
      )

    def wait_write(sb):
      pltpu.make_async_copy(
          bufb.at[sb], out_hbm.at[0, :, pl.ds(b0, B_PER_W)], wsems[sb]
      ).wait()

    def iteration(s, sb, first, do_issue, do_idx):
      if not first:
        wait_write(sb)
      for blk in range(NBLK):
        wait_gather(blk)
        transpose(blk, sb)
        if do_issue:
          gather(s + 1, sb ^ 1, blk)
      if do_idx:
        build_idx(s + 2, sb)
      write(s, sb)

    build_idx(0, 0)
    build_idx(1, 1)
    for blk in range(NBLK):
      gather(0, 0, blk)

    iteration(0, 0, True, True, True)
    iteration(1, 1, True, True, True)

    def pair(p, carry):
      s = 2 * p
      iteration(s, 0, False, True, True)
      iteration(s + 1, 1, False, True, True)
      return carry

    lax.fori_loop(1, SEQ // 2 - 1, pair, 0)

    iteration(SEQ - 2, 0, False, True, False)
    iteration(SEQ - 1, 1, False, False, False)

    wait_write(0)
    wait_write(1)

  return body(table, idx_flat)


def kernel(x, table):
  idx_flat = x.reshape(-1).astype(jnp.int32)
  out_t = _sc_gather_t(table, idx_flat)
  return jnp.transpose(out_t, (2, 0, 1))


# speedup vs baseline: 1.2101x; 1.2101x over previous
import functools

import jax
import jax.numpy as jnp
from jax import lax
from jax.experimental import pallas as pl
from jax.experimental.pallas import tpu as pltpu
from jax.experimental.pallas import tpu_sc as plsc

VOCAB_ROWS = 1_000_000
EMB_DIM = 64
SEQ = 50
BATCH = 16384
NUM_CORES = 2
NUM_SUBCORES = 16
NUM_WORKERS = NUM_CORES * NUM_SUBCORES
B_PER_W = BATCH // NUM_WORKERS
CHUNK = 128
NBLK = B_PER_W // CHUNK

TP_COLS = 2048
TP2_ROWS = 2048


def _tp_body(in_ref, out_ref):
  out_ref[...] = in_ref[...].T


def _tc_transpose_table(table_t):
  return pl.pallas_call(
      _tp_body,
      grid=(pl.cdiv(VOCAB_ROWS, TP_COLS),),
      in_specs=[pl.BlockSpec((EMB_DIM, TP_COLS), lambda i: (0, i))],
      out_specs=pl.BlockSpec((TP_COLS, EMB_DIM), lambda i: (i, 0)),
      out_shape=jax.ShapeDtypeStruct((VOCAB_ROWS, EMB_DIM), jnp.float32),
  )(table_t)


def _tp2_body(in_ref, out_ref):
  out_ref[0] = in_ref[0].T


def _tc_transpose_out(g):
  return pl.pallas_call(
      _tp2_body,
      grid=(SEQ, BATCH // TP2_ROWS),
      in_specs=[pl.BlockSpec((1, TP2_ROWS, EMB_DIM), lambda s, j: (s, j, 0))],
      out_specs=pl.BlockSpec((1, EMB_DIM, TP2_ROWS), lambda s, j: (s, 0, j)),
      out_shape=jax.ShapeDtypeStruct((SEQ, EMB_DIM, BATCH), jnp.float32),
  )(g)


@jax.jit
def _sc_gather_s(table, idx_flat):
  mesh = plsc.VectorSubcoreMesh(core_axis_name="c", subcore_axis_name="s")

  @functools.partial(
      pl.kernel,
      out_type=jax.ShapeDtypeStruct((SEQ, BATCH, EMB_DIM), jnp.float32),
      mesh=mesh,
      scratch_types=[
          pltpu.VMEM((B_PER_W * SEQ,), jnp.int32),
          pltpu.VMEM((2, B_PER_W), jnp.int32),
          pltpu.VMEM((2, NBLK, CHUNK, EMB_DIM), jnp.float32),
          [pltpu.SemaphoreType.DMA] * (2 * NBLK),
          [pltpu.SemaphoreType.DMA] * (2 * NBLK),
      ],
      compiler_params=pltpu.CompilerParams(
          use_tc_tiling_on_sc=False, needs_layout_passes=False
      ),
  )
  def body(table_hbm, idx_hbm, out_hbm, x_v, idx_v, bufs, gsems, wsems):
    wid = lax.axis_index("s") * NUM_CORES + lax.axis_index("c")
    b0 = wid * B_PER_W
    pltpu.sync_copy(idx_hbm.at[pl.ds(b0 * SEQ, B_PER_W * SEQ)], x_v)

    iota = lax.iota(jnp.int32, 16)
    iota_s = iota * SEQ

    def build_idx(s, slot):
      def step(g, carry):
        v = plsc.load_gather(x_v, [iota_s + (g * 16 * SEQ + s)])
        idx_v[slot, pl.ds(g * 16, 16)] = v
        return carry
      lax.fori_loop(0, B_PER_W // 16, step, 0)

    def gather(s, slot, bank, blk):
      pltpu.async_copy(
          table_hbm.at[idx_v.at[slot].at[pl.ds(blk * CHUNK, CHUNK)]],
          bufs.at[bank, blk],
          gsems[bank * NBLK + blk],
      )

    def wait_gather(bank, blk):
      pltpu.make_async_copy(
          table_hbm.at[idx_v.at[0].at[pl.ds(0, CHUNK)]],
          bufs.at[bank, blk],
          gsems[bank * NBLK + blk],
      ).wait()

    def write(s, bank, blk):
      pltpu.async_copy(
          bufs.at[bank, blk],
          out_hbm.at[s, pl.ds(b0 + blk * CHUNK, CHUNK), :],
          wsems[bank * NBLK + blk],
      )

    def wait_write(bank, blk):
      pltpu.make_async_copy(
          bufs.at[bank, blk],
          out_hbm.at[0, pl.ds(b0, CHUNK), :],
          wsems[bank * NBLK + blk],
      ).wait()

    def iteration(s, sb, first, do_issue, do_idx):
      for blk in range(NBLK):
        wait_gather(sb, blk)
        write(s, sb, blk)
        if do_issue:
          if not first:
            wait_write(sb ^ 1, blk)
          gather(s + 1, sb ^ 1, sb ^ 1, blk)
      if do_idx:
        build_idx(s + 2, sb)

    build_idx(0, 0)
    build_idx(1, 1)
    for blk in range(NBLK):
      gather(0, 0, 0, blk)

    iteration(0, 0, True, True, True)
    iteration(1, 1, False, True, True)

    def pair(p, carry):
      s = 2 * p
      iteration(s, 0, False, True, True)
      iteration(s + 1, 1, False, True, True)
      return carry

    lax.fori_loop(1, SEQ // 2 - 1, pair, 0)

    iteration(SEQ - 2, 0, False, True, False)
    iteration(SEQ - 1, 1, False, False, False)

    for blk in range(NBLK):
      wait_write(0, blk)
      wait_write(1, blk)

  return body(table, idx_flat)


def kernel(x, table):
  idx_flat = x.reshape(-1).astype(jnp.int32)
  table_rm = _tc_transpose_table(table.T)
  g = _sc_gather_s(table_rm, idx_flat)
  out_t = _tc_transpose_out(g)
  return jnp.transpose(out_t, (2, 0, 1))
